# Initial kernel scaffold; baseline (speedup 1.0000x reference)
import jax
import jax.numpy as jnp
from jax.experimental import pallas as pl


def kernel(m_emb, edge_index, eig, hg_pos_v, hg_pos_e, hg_neg_v, hg_neg_e, W1, b1, W2, b2, W3, b3, W_dgn, b_dgn, theta1_W, theta1_b, theta2_W, theta2_b):
    raise NotImplementedError("write your pallas kernel here")


# speedup vs baseline: 1.1442x; 1.1442x over previous
import functools

import jax
import jax.numpy as jnp
from jax import lax
from jax.experimental import pallas as pl
from jax.experimental.pallas import tpu as pltpu

N = 10000
E = 320000
D_IN = 2816
D = 128
NHE = 2000
P = 40000
DELTA = 2.5


_MLP_BN = 400


def _mlp_body(m_ref, w1_ref, b1_ref, w2_ref, b2_ref, w3_ref, b3_ref,
              x_ref, xs_ref):
    h = jnp.maximum(
        jnp.dot(m_ref[...], w1_ref[...], preferred_element_type=jnp.float32)
        + b1_ref[...], 0.0)
    h = jnp.maximum(
        jnp.dot(h, w2_ref[...], preferred_element_type=jnp.float32)
        + b2_ref[...], 0.0)
    x = jnp.dot(h, w3_ref[...], preferred_element_type=jnp.float32) + b3_ref[...]
    x_ref[...] = x
    xs_ref[...] = jnp.stack([x[:, :64], x[:, 64:]])


def _mlp_tc(m_emb, W1, b1, W2, b2, W3, b3):
    grid = N // _MLP_BN
    return pl.pallas_call(
        _mlp_body,
        grid=(grid,),
        in_specs=[
            pl.BlockSpec((_MLP_BN, D_IN), lambda i: (i, 0)),
            pl.BlockSpec((D_IN, 512), lambda i: (0, 0)),
            pl.BlockSpec((512,), lambda i: (0,)),
            pl.BlockSpec((512, 512), lambda i: (0, 0)),
            pl.BlockSpec((512,), lambda i: (0,)),
            pl.BlockSpec((512, D), lambda i: (0, 0)),
            pl.BlockSpec((D,), lambda i: (0,)),
        ],
        out_specs=[
            pl.BlockSpec((_MLP_BN, D), lambda i: (i, 0)),
            pl.BlockSpec((2, _MLP_BN, 64), lambda i: (0, i, 0)),
        ],
        out_shape=[
            jax.ShapeDtypeStruct((N, D), jnp.float32),
            jax.ShapeDtypeStruct((2, N, 64), jnp.float32),
        ],
    )(m_emb, W1, b1, W2, b2, W3, b3)


def _dgn_jnp(x, src, dst, eig, W, b):
    n, d = x.shape
    msg = x[src]
    ones = jnp.ones(src.shape[0], dtype=x.dtype)
    deg = jnp.zeros((n,), x.dtype).at[dst].add(ones)
    agg_sum = jnp.zeros((n, d), x.dtype).at[dst].add(msg)
    e = eig[dst, 2] - eig[src, 2]
    abs_e = jnp.abs(e)
    denom = jnp.zeros((n,), x.dtype).at[dst].add(abs_e) + 1e-8
    w_av = abs_e / denom[dst]
    agg_av = jnp.zeros((n, d), x.dtype).at[dst].add(msg * w_av[:, None])
    w_dx = e / denom[dst]
    sum_wdx = jnp.zeros((n,), x.dtype).at[dst].add(w_dx)
    agg_dx = jnp.zeros((n, d), x.dtype).at[dst].add(msg * w_dx[:, None]) - x * sum_wdx[:, None]
    h = jnp.concatenate([agg_av, agg_dx, agg_sum], axis=-1)
    amp = jnp.log(deg + 1.0) / DELTA
    h = jnp.concatenate([h, h * amp[:, None]], axis=-1)
    return h @ W + b


def _hgnn_jnp(X, v, e, n_he, is_last):
    n = N
    d = X.shape[1]
    one = jnp.ones(v.shape[0], X.dtype)
    cnt_e = jnp.maximum(jnp.zeros((n_he,), X.dtype).at[e].add(one), 1.0)
    Y = jnp.zeros((n_he, d), X.dtype).at[e].add(X[v]) / cnt_e[:, None]
    cnt_v = jnp.maximum(jnp.zeros((n,), X.dtype).at[v].add(one), 1.0)
    Xo = jnp.zeros((n, d), X.dtype).at[v].add(Y[e]) / cnt_v[:, None]
    if not is_last:
        Xo = jax.nn.relu(Xo)
    return Xo, Y


def kernel(m_emb, edge_index, eig, hg_pos_v, hg_pos_e, hg_neg_v, hg_neg_e,
           W1, b1, W2, b2, W3, b3, W_dgn, b_dgn,
           theta1_W, theta1_b, theta2_W, theta2_b):
    x, _xs = _mlp_tc(m_emb, W1, b1, W2, b2, W3, b3)
    src = edge_index[0].astype(jnp.int32)
    dst = edge_index[1].astype(jnp.int32)
    x = _dgn_jnp(x, src, dst, eig, W_dgn, b_dgn)

    pv = hg_pos_v.astype(jnp.int32)
    pe = hg_pos_e.astype(jnp.int32)
    nv = hg_neg_v.astype(jnp.int32)
    ne = hg_neg_e.astype(jnp.int32)

    X = x @ theta1_W + theta1_b
    X1, _ = _hgnn_jnp(X, pv, pe, NHE, False)
    X2, _ = _hgnn_jnp(X, nv, ne, NHE, False)
    X1p = X1 @ theta2_W + theta2_b
    X2p = X2 @ theta2_W + theta2_b
    X1, Yp = _hgnn_jnp(X1p, pv, pe, NHE, True)
    X2, Yn = _hgnn_jnp(X2p, nv, ne, NHE, True)
    return (X1, X2, Yp, Yn)


# speedup vs baseline: 6.2030x; 5.4211x over previous
import jax
import jax.numpy as jnp
from jax import lax
from jax.experimental import pallas as pl
from jax.experimental.pallas import tpu as pltpu
from jax.experimental.pallas import tpu_sc as plsc

N = 10000
E = 320000
D_IN = 2816
D = 128
NHE = 2000
P = 40000
DELTA = 2.5

_SC_MESH = plsc.VectorSubcoreMesh(core_axis_name="c", subcore_axis_name="s")
_SC_PARAMS = pltpu.CompilerParams(use_tc_tiling_on_sc=False,
                                  needs_layout_passes=False)

_F32 = jnp.float32
_I32 = jnp.int32


def _lane_consts():
    io = lax.iota(_I32, 16)
    ones_row = jnp.where(io == 0, 1.0, 0.0).astype(_F32)
    deg_row = jnp.where(io == 1, 1.0, 0.0).astype(_F32)
    e2_row = jnp.where(io == 2, 1.0, 0.0).astype(_F32)
    return ones_row, deg_row, e2_row


def _bcast(buf, j):
    return plsc.load_gather(buf, [jnp.full((16,), j, _I32)])


def _zero_vmem(buf, rows, width):
    z = jnp.zeros((16,), _F32)

    @pl.loop(0, rows)
    def _(r):
        for f in range(0, width, 16):
            buf[r, pl.ds(f, 16)] = z


_MLP_BN = 400


def _mlp_body(m_ref, w1_ref, b1_ref, w2_ref, b2_ref, w3_ref, b3_ref,
              x_ref, xq_ref):
    h = jnp.maximum(
        jnp.dot(m_ref[...], w1_ref[...], preferred_element_type=_F32)
        + b1_ref[...], 0.0)
    h = jnp.maximum(
        jnp.dot(h, w2_ref[...], preferred_element_type=_F32)
        + b2_ref[...], 0.0)
    x = jnp.dot(h, w3_ref[...], preferred_element_type=_F32) + b3_ref[...]
    x_ref[...] = x
    xq_ref[...] = jnp.stack([x[:, 0:32], x[:, 32:64], x[:, 64:96], x[:, 96:128]])


def _mlp_tc(m_emb, W1, b1, W2, b2, W3, b3):
    grid = N // _MLP_BN
    return pl.pallas_call(
        _mlp_body,
        grid=(grid,),
        in_specs=[
            pl.BlockSpec((_MLP_BN, D_IN), lambda i: (i, 0)),
            pl.BlockSpec((D_IN, 512), lambda i: (0, 0)),
            pl.BlockSpec((512,), lambda i: (0,)),
            pl.BlockSpec((512, 512), lambda i: (0, 0)),
            pl.BlockSpec((512,), lambda i: (0,)),
            pl.BlockSpec((512, D), lambda i: (0, 0)),
            pl.BlockSpec((D,), lambda i: (0,)),
        ],
        out_specs=[
            pl.BlockSpec((_MLP_BN, D), lambda i: (i, 0)),
            pl.BlockSpec((4, _MLP_BN, 32), lambda i: (0, i, 0)),
        ],
        out_shape=[
            jax.ShapeDtypeStruct((N, D), _F32),
            jax.ShapeDtypeStruct((4, N, 32), _F32),
        ],
    )(m_emb, W1, b1, W2, b2, W3, b3)


_PK = 128
_PNC = E // _PK
_HK = 80
_HNC = P // _HK


def _prep_body(src_hbm, dst_hbm, eig2_hbm, pv_hbm, pe_hbm, nv_hbm, ne_hbm,
               ea_hbm, ee_hbm, scal_hbm, cvp_hbm, cep_hbm, cvn_hbm, cen_hbm,
               scal_sp, cvp_sp, cep_sp, cvn_sp, cen_sp,
               eig2_v, isrc, idst, abuf, ebuf, rowbuf, iv, ie, onesbuf,
               zb, sem):
    c = lax.axis_index("c")
    sid = lax.axis_index("s")
    g = c * 16 + sid
    ones_row, deg_row, e2_row = _lane_consts()

    _zero_vmem(zb, 125, 16)

    @pl.loop(0, 5)
    def _(k):
        rr = sid * 625 + k * 125
        pltpu.sync_copy(zb, scal_sp.at[pl.ds(rr, 125)])
        pltpu.sync_copy(zb, cvp_sp.at[pl.ds(rr, 125)])
        pltpu.sync_copy(zb, cvn_sp.at[pl.ds(rr, 125)])

    pltpu.sync_copy(zb, cep_sp.at[pl.ds(sid * 125, 125)])
    pltpu.sync_copy(zb, cen_sp.at[pl.ds(sid * 125, 125)])

    @pl.loop(0, _HK)
    def _(r):
        onesbuf[r, pl.ds(0, 16)] = ones_row

    pltpu.sync_copy(eig2_hbm, eig2_v)
    plsc.subcore_barrier()

    @pl.loop(g, _PNC, step=32)
    def _(ch):
        base = ch * _PK
        pltpu.sync_copy(src_hbm.at[pl.ds(base, _PK)], isrc)
        pltpu.sync_copy(dst_hbm.at[pl.ds(base, _PK)], idst)

        @pl.loop(0, _PK, step=16)
        def _(j):
            s16 = isrc[pl.ds(j, 16)]
            d16 = idst[pl.ds(j, 16)]
            ev = plsc.load_gather(eig2_v, [d16]) - plsc.load_gather(eig2_v, [s16])
            ebuf[pl.ds(j, 16)] = ev
            abuf[pl.ds(j, 16)] = jnp.abs(ev)

        @pl.loop(0, _PK)
        def _(j):
            fa = _bcast(abuf, j)
            fe = _bcast(ebuf, j)
            rowbuf[j, pl.ds(0, 16)] = ones_row + fa * deg_row + fe * e2_row

        pltpu.sync_copy(abuf, ea_hbm.at[pl.ds(base, _PK)])
        pltpu.sync_copy(ebuf, ee_hbm.at[pl.ds(base, _PK)])
        pltpu.sync_copy(rowbuf, scal_sp.at[idst], add=True)

    @pl.loop(g, _HNC, step=32)
    def _(ch):
        base = ch * _HK
        pltpu.sync_copy(pv_hbm.at[pl.ds(base, _HK)], iv)
        pltpu.sync_copy(pe_hbm.at[pl.ds(base, _HK)], ie)
        d1 = pltpu.async_copy(onesbuf, cvp_sp.at[iv], sem, add=True)
        d2 = pltpu.async_copy(onesbuf, cep_sp.at[ie], sem, add=True)
        d1.wait()
        d2.wait()
        pltpu.sync_copy(nv_hbm.at[pl.ds(base, _HK)], iv)
        pltpu.sync_copy(ne_hbm.at[pl.ds(base, _HK)], ie)
        d3 = pltpu.async_copy(onesbuf, cvn_sp.at[iv], sem, add=True)
        d4 = pltpu.async_copy(onesbuf, cen_sp.at[ie], sem, add=True)
        d3.wait()
        d4.wait()

    plsc.subcore_barrier()

    @pl.loop(0, 5)
    def _(k):
        rr = sid * 625 + k * 125
        orow = c * N + rr
        pltpu.sync_copy(scal_sp.at[pl.ds(rr, 125)], zb)
        pltpu.sync_copy(zb, scal_hbm.at[pl.ds(orow, 125)])
        pltpu.sync_copy(cvp_sp.at[pl.ds(rr, 125)], zb)
        pltpu.sync_copy(zb, cvp_hbm.at[pl.ds(orow, 125)])
        pltpu.sync_copy(cvn_sp.at[pl.ds(rr, 125)], zb)
        pltpu.sync_copy(zb, cvn_hbm.at[pl.ds(orow, 125)])

    re = sid * 125
    oe = c * NHE + re
    pltpu.sync_copy(cep_sp.at[pl.ds(re, 125)], zb)
    pltpu.sync_copy(zb, cep_hbm.at[pl.ds(oe, 125)])
    pltpu.sync_copy(cen_sp.at[pl.ds(re, 125)], zb)
    pltpu.sync_copy(zb, cen_hbm.at[pl.ds(oe, 125)])


def _prep_sc(src, dst, eig2, pv, pe, nv, ne):
    fn = pl.kernel(
        _prep_body,
        out_type=(
            jax.ShapeDtypeStruct((E,), _F32),
            jax.ShapeDtypeStruct((E,), _F32),
            jax.ShapeDtypeStruct((2 * N, 16), _F32),
            jax.ShapeDtypeStruct((2 * N, 16), _F32),
            jax.ShapeDtypeStruct((2 * NHE, 16), _F32),
            jax.ShapeDtypeStruct((2 * N, 16), _F32),
            jax.ShapeDtypeStruct((2 * NHE, 16), _F32),
        ),
        mesh=_SC_MESH,
        scratch_types=[
            pltpu.VMEM_SHARED((N, 16), _F32),
            pltpu.VMEM_SHARED((N, 16), _F32),
            pltpu.VMEM_SHARED((NHE, 16), _F32),
            pltpu.VMEM_SHARED((N, 16), _F32),
            pltpu.VMEM_SHARED((NHE, 16), _F32),
            pltpu.VMEM((N,), _F32),
            pltpu.VMEM((_PK,), _I32),
            pltpu.VMEM((_PK,), _I32),
            pltpu.VMEM((_PK,), _F32),
            pltpu.VMEM((_PK,), _F32),
            pltpu.VMEM((_PK, 16), _F32),
            pltpu.VMEM((_HK,), _I32),
            pltpu.VMEM((_HK,), _I32),
            pltpu.VMEM((_HK, 16), _F32),
            pltpu.VMEM((125, 16), _F32),
            pltpu.SemaphoreType.DMA,
        ],
        compiler_params=_SC_PARAMS,
    )
    return fn(src, dst, eig2, pv, pe, nv, ne)


_K = 128
_NCHUNK = E // _K
_RPS = N // 16
_RC = 125


def _dgn_sc_body(xq_hbm, src_hbm, dst_hbm, ea_hbm, ee_hbm,
                 s1_hbm, s2_hbm, s3_hbm,
                 s1_sp, s2_sp, s3_sp,
                 isrc, idst, igat, msg, mA, mD, abuf, ebuf, zb, sem):
    c = lax.axis_index("c")
    sid = lax.axis_index("s")
    r0 = sid * _RPS

    for p in range(2):
        q = 2 * c + p
        coff = q * N
        _zero_vmem(zb, _RC, 32)

        @pl.loop(0, 5)
        def _(k):
            rr = r0 + k * _RC
            pltpu.sync_copy(zb, s1_sp.at[pl.ds(rr, _RC)])
            pltpu.sync_copy(zb, s2_sp.at[pl.ds(rr, _RC)])
            pltpu.sync_copy(zb, s3_sp.at[pl.ds(rr, _RC)])

        plsc.subcore_barrier()

        @pl.loop(sid, _NCHUNK, step=16)
        def _(ch):
            base = ch * _K
            pltpu.sync_copy(src_hbm.at[pl.ds(base, _K)], isrc)
            pltpu.sync_copy(dst_hbm.at[pl.ds(base, _K)], idst)
            pltpu.sync_copy(ea_hbm.at[pl.ds(base, _K)], abuf)
            pltpu.sync_copy(ee_hbm.at[pl.ds(base, _K)], ebuf)

            @pl.loop(0, _K, step=16)
            def _(j):
                igat[pl.ds(j, 16)] = isrc[pl.ds(j, 16)] + coff

            pltpu.async_copy(xq_hbm.at[igat], msg, sem).wait()

            @pl.loop(0, _K)
            def _(j):
                fa = _bcast(abuf, j)
                fe = _bcast(ebuf, j)
                for f in range(2):
                    m = msg[j, pl.ds(f * 16, 16)]
                    mA[j, pl.ds(f * 16, 16)] = m * fa
                    mD[j, pl.ds(f * 16, 16)] = m * fe

            d1 = pltpu.async_copy(msg, s1_sp.at[idst], sem, add=True)
            d2 = pltpu.async_copy(mA, s2_sp.at[idst], sem, add=True)
            d3 = pltpu.async_copy(mD, s3_sp.at[idst], sem, add=True)
            d1.wait()
            d2.wait()
            d3.wait()

        plsc.subcore_barrier()

        @pl.loop(0, 5)
        def _(k):
            rr = r0 + k * _RC
            orow = coff + rr
            pltpu.sync_copy(s1_sp.at[pl.ds(rr, _RC)], zb)
            pltpu.sync_copy(zb, s1_hbm.at[pl.ds(orow, _RC)])
            pltpu.sync_copy(s2_sp.at[pl.ds(rr, _RC)], zb)
            pltpu.sync_copy(zb, s2_hbm.at[pl.ds(orow, _RC)])
            pltpu.sync_copy(s3_sp.at[pl.ds(rr, _RC)], zb)
            pltpu.sync_copy(zb, s3_hbm.at[pl.ds(orow, _RC)])

        plsc.subcore_barrier()


def _dgn_sc(xq, src, dst, ea, ee):
    fn = pl.kernel(
        _dgn_sc_body,
        out_type=(
            jax.ShapeDtypeStruct((4 * N, 32), _F32),
            jax.ShapeDtypeStruct((4 * N, 32), _F32),
            jax.ShapeDtypeStruct((4 * N, 32), _F32),
        ),
        mesh=_SC_MESH,
        scratch_types=[
            pltpu.VMEM_SHARED((N, 32), _F32),
            pltpu.VMEM_SHARED((N, 32), _F32),
            pltpu.VMEM_SHARED((N, 32), _F32),
            pltpu.VMEM((_K,), _I32),
            pltpu.VMEM((_K,), _I32),
            pltpu.VMEM((_K,), _I32),
            pltpu.VMEM((_K, 32), _F32),
            pltpu.VMEM((_K, 32), _F32),
            pltpu.VMEM((_K, 32), _F32),
            pltpu.VMEM((_K,), _F32),
            pltpu.VMEM((_K,), _F32),
            pltpu.VMEM((_RC, 32), _F32),
            pltpu.SemaphoreType.DMA,
        ],
        compiler_params=_SC_PARAMS,
    )
    return fn(xq, src, dst, ea, ee)


def _inv_body(cvp_ref, cep_ref, cvn_ref, cen_ref,
              ivp_ref, iep_ref, ivn_ref, ien_ref):
    def inv(ref, n):
        a = ref[...]
        r = 1.0 / jnp.maximum(a[:n, 0] + a[n:, 0], 1.0)
        return jnp.broadcast_to(r[:, None], (n, 16))

    ivp_ref[...] = inv(cvp_ref, N)
    iep_ref[...] = inv(cep_ref, NHE)
    ivn_ref[...] = inv(cvn_ref, N)
    ien_ref[...] = inv(cen_ref, NHE)


def _inv_tc(cvp, cep, cvn, cen):
    return pl.pallas_call(
        _inv_body,
        out_shape=[
            jax.ShapeDtypeStruct((N, 16), _F32),
            jax.ShapeDtypeStruct((NHE, 16), _F32),
            jax.ShapeDtypeStruct((N, 16), _F32),
            jax.ShapeDtypeStruct((NHE, 16), _F32),
        ],
    )(cvp, cep, cvn, cen)


_FB = 400


def _finish_body(s1q0, s1q1, s1q2, s1q3, s2q0, s2q1, s2q2, s2q3,
                 s3q0, s3q1, s3q2, s3q3, sca, scb,
                 x_ref, wd_ref, bd_ref, t1w_ref, t1b_ref, xs_ref):
    s1 = jnp.concatenate(
        [s1q0[...], s1q1[...], s1q2[...], s1q3[...]], axis=-1)
    s2 = jnp.concatenate(
        [s2q0[...], s2q1[...], s2q2[...], s2q3[...]], axis=-1)
    s3 = jnp.concatenate(
        [s3q0[...], s3q1[...], s3q2[...], s3q3[...]], axis=-1)
    sc = sca[...] + scb[...]
    deg = sc[:, 0]
    s4 = sc[:, 1]
    s5 = sc[:, 2]
    x = x_ref[...]
    inv = 1.0 / (s4 + 1e-8)
    agg_av = s2 * inv[:, None]
    agg_dx = s3 * inv[:, None] - x * (s5 * inv)[:, None]
    amp = jnp.log(deg + 1.0) * (1.0 / DELTA)
    h = jnp.concatenate([agg_av, agg_dx, s1], axis=-1)
    h6 = jnp.concatenate([h, h * amp[:, None]], axis=-1)
    xd = jnp.dot(h6, wd_ref[...], preferred_element_type=_F32) + bd_ref[...]
    X = jnp.dot(xd, t1w_ref[...], preferred_element_type=_F32) + t1b_ref[...]
    xs_ref[...] = jnp.stack([X[:, :64], X[:, 64:]])


def _finish_tc(s1o, s2o, s3o, scal, x, W_dgn, b_dgn, t1w, t1b):
    grid = N // _FB
    off = N // _FB

    def qspec(q):
        return pl.BlockSpec((_FB, 32), lambda i, q=q: (i + q * off, 0))

    return pl.pallas_call(
        _finish_body,
        grid=(grid,),
        in_specs=[
            qspec(0), qspec(1), qspec(2), qspec(3),
            qspec(0), qspec(1), qspec(2), qspec(3),
            qspec(0), qspec(1), qspec(2), qspec(3),
            pl.BlockSpec((_FB, 16), lambda i: (i, 0)),
            pl.BlockSpec((_FB, 16), lambda i: (i + off, 0)),
            pl.BlockSpec((_FB, D), lambda i: (i, 0)),
            pl.BlockSpec((6 * D, D), lambda i: (0, 0)),
            pl.BlockSpec((D,), lambda i: (0,)),
            pl.BlockSpec((D, D), lambda i: (0, 0)),
            pl.BlockSpec((D,), lambda i: (0,)),
        ],
        out_specs=pl.BlockSpec((2, _FB, 64), lambda i: (0, i, 0)),
        out_shape=jax.ShapeDtypeStruct((2, N, 64), _F32),
    )(s1o, s1o, s1o, s1o, s2o, s2o, s2o, s2o, s3o, s3o, s3o, s3o,
      scal, scal, x, W_dgn, b_dgn, t1w, t1b)


_YRPS = NHE // 16
_XRPS = N // 16


def _hg_body(relu,
             xp_hbm, xn_hbm, pv_hbm, pe_hbm, nv_hbm, ne_hbm,
             iep_hbm, ivp_hbm, ien_hbm, ivn_hbm,
             xop_hbm, xon_hbm, yp_hbm, yn_hbm,
             yp_sp, yn_sp, xop_sp, xon_sp,
             iv, ie, igat, buf, invb, zb, sem):
    c = lax.axis_index("c")
    sid = lax.axis_index("s")
    _zero_vmem(zb, _RC, 64)
    ry = sid * _YRPS
    rx = sid * _XRPS

    pltpu.sync_copy(zb.at[pl.ds(0, _YRPS)], yp_sp.at[pl.ds(ry, _YRPS)])
    pltpu.sync_copy(zb.at[pl.ds(0, _YRPS)], yn_sp.at[pl.ds(ry, _YRPS)])

    @pl.loop(0, 5)
    def _(k):
        rr = rx + k * _RC
        pltpu.sync_copy(zb, xop_sp.at[pl.ds(rr, _RC)])
        pltpu.sync_copy(zb, xon_sp.at[pl.ds(rr, _RC)])

    plsc.subcore_barrier()

    coff = c * N

    for (v_hbm, e_hbm, x_hbm, y_sp) in (
            (pv_hbm, pe_hbm, xp_hbm, yp_sp),
            (nv_hbm, ne_hbm, xn_hbm, yn_sp)):
        @pl.loop(sid, _HNC, step=16)
        def _(ch):
            base = ch * _HK
            pltpu.sync_copy(v_hbm.at[pl.ds(base, _HK)], iv)
            pltpu.sync_copy(e_hbm.at[pl.ds(base, _HK)], ie)

            @pl.loop(0, _HK, step=16)
            def _(j):
                igat[pl.ds(j, 16)] = iv[pl.ds(j, 16)] + coff

            pltpu.async_copy(x_hbm.at[igat], buf, sem).wait()
            pltpu.sync_copy(buf, y_sp.at[ie], add=True)

    plsc.subcore_barrier()

    for (ie_hbm, y_sp, y_hbm) in ((iep_hbm, yp_sp, yp_hbm),
                                  (ien_hbm, yn_sp, yn_hbm)):
        pltpu.sync_copy(y_sp.at[pl.ds(ry, _YRPS)], zb.at[pl.ds(0, _YRPS)])
        pltpu.sync_copy(ie_hbm.at[pl.ds(ry, _YRPS)], invb)

        @pl.loop(0, _YRPS)
        def _(r):
            fi = invb[r, pl.ds(0, 16)]
            for f in range(4):
                zb[r, pl.ds(f * 16, 16)] = zb[r, pl.ds(f * 16, 16)] * fi

        pltpu.sync_copy(zb.at[pl.ds(0, _YRPS)], y_hbm.at[pl.ds(c * NHE + ry, _YRPS)])

    plsc.subcore_barrier()

    yoff = c * NHE
    for (v_hbm, e_hbm, y_hbm, xo_sp) in (
            (pv_hbm, pe_hbm, yp_hbm, xop_sp),
            (nv_hbm, ne_hbm, yn_hbm, xon_sp)):
        @pl.loop(sid, _HNC, step=16)
        def _(ch):
            base = ch * _HK
            pltpu.sync_copy(v_hbm.at[pl.ds(base, _HK)], iv)
            pltpu.sync_copy(e_hbm.at[pl.ds(base, _HK)], ie)

            @pl.loop(0, _HK, step=16)
            def _(j):
                igat[pl.ds(j, 16)] = ie[pl.ds(j, 16)] + yoff

            pltpu.async_copy(y_hbm.at[igat], buf, sem).wait()
            pltpu.sync_copy(buf, xo_sp.at[iv], add=True)

    plsc.subcore_barrier()

    for (iv_hbm, xo_sp, xo_hbm) in ((ivp_hbm, xop_sp, xop_hbm),
                                    (ivn_hbm, xon_sp, xon_hbm)):
        @pl.loop(0, 5)
        def _(k):
            rr = rx + k * _RC
            pltpu.sync_copy(xo_sp.at[pl.ds(rr, _RC)], zb)
            pltpu.sync_copy(iv_hbm.at[pl.ds(rr, _RC)], invb)

            @pl.loop(0, _RC)
            def _(r):
                fi = invb[r, pl.ds(0, 16)]
                for f in range(4):
                    val = zb[r, pl.ds(f * 16, 16)] * fi
                    if relu:
                        val = jnp.maximum(val, 0.0)
                    zb[r, pl.ds(f * 16, 16)] = val

            pltpu.sync_copy(zb, xo_hbm.at[pl.ds(coff + rr, _RC)])


def _hg_sc(relu, xp, xn, pv, pe, nv, ne, iep, ivp, ien, ivn):
    def body(*refs):
        return _hg_body(relu, *refs)

    fn = pl.kernel(
        body,
        out_type=(
            jax.ShapeDtypeStruct((2 * N, 64), _F32),
            jax.ShapeDtypeStruct((2 * N, 64), _F32),
            jax.ShapeDtypeStruct((2 * NHE, 64), _F32),
            jax.ShapeDtypeStruct((2 * NHE, 64), _F32),
        ),
        mesh=_SC_MESH,
        scratch_types=[
            pltpu.VMEM_SHARED((NHE, 64), _F32),
            pltpu.VMEM_SHARED((NHE, 64), _F32),
            pltpu.VMEM_SHARED((N, 64), _F32),
            pltpu.VMEM_SHARED((N, 64), _F32),
            pltpu.VMEM((_HK,), _I32),
            pltpu.VMEM((_HK,), _I32),
            pltpu.VMEM((_HK,), _I32),
            pltpu.VMEM((_HK, 64), _F32),
            pltpu.VMEM((_RC, 16), _F32),
            pltpu.VMEM((_RC, 64), _F32),
            pltpu.SemaphoreType.DMA,
        ],
        compiler_params=_SC_PARAMS,
    )
    return fn(xp, xn, pv, pe, nv, ne, iep, ivp, ien, ivn)


def _theta2_body(xpa, xpb, xna, xnb, w_ref, b_ref, op_ref, on_ref):
    w = w_ref[...]
    b = b_ref[...]
    Xp = jnp.concatenate([xpa[...], xpb[...]], axis=-1)
    Xn = jnp.concatenate([xna[...], xnb[...]], axis=-1)
    Yp = jnp.dot(Xp, w, preferred_element_type=_F32) + b
    Yn = jnp.dot(Xn, w, preferred_element_type=_F32) + b
    op_ref[...] = jnp.stack([Yp[:, :64], Yp[:, 64:]])
    on_ref[...] = jnp.stack([Yn[:, :64], Yn[:, 64:]])


def _theta2_tc(xop, xon, t2w, t2b):
    grid = N // _FB
    off = N // _FB
    return pl.pallas_call(
        _theta2_body,
        grid=(grid,),
        in_specs=[
            pl.BlockSpec((_FB, 64), lambda i: (i, 0)),
            pl.BlockSpec((_FB, 64), lambda i: (i + off, 0)),
            pl.BlockSpec((_FB, 64), lambda i: (i, 0)),
            pl.BlockSpec((_FB, 64), lambda i: (i + off, 0)),
            pl.BlockSpec((D, D), lambda i: (0, 0)),
            pl.BlockSpec((D,), lambda i: (0,)),
        ],
        out_specs=[
            pl.BlockSpec((2, _FB, 64), lambda i: (0, i, 0)),
            pl.BlockSpec((2, _FB, 64), lambda i: (0, i, 0)),
        ],
        out_shape=[
            jax.ShapeDtypeStruct((2, N, 64), _F32),
            jax.ShapeDtypeStruct((2, N, 64), _F32),
        ],
    )(xop, xop, xon, xon, t2w, t2b)


def kernel(m_emb, edge_index, eig, hg_pos_v, hg_pos_e, hg_neg_v, hg_neg_e,
           W1, b1, W2, b2, W3, b3, W_dgn, b_dgn,
           theta1_W, theta1_b, theta2_W, theta2_b):
    src = edge_index[0].astype(_I32)
    dst = edge_index[1].astype(_I32)
    eig2 = jnp.asarray(eig[:, 2], _F32)
    pv = hg_pos_v.astype(_I32)
    pe = hg_pos_e.astype(_I32)
    nv = hg_neg_v.astype(_I32)
    ne = hg_neg_e.astype(_I32)

    ea, ee, scal, cvp, cep, cvn, cen = _prep_sc(src, dst, eig2, pv, pe, nv, ne)
    x, xq = _mlp_tc(m_emb, W1, b1, W2, b2, W3, b3)
    xq2 = xq.reshape(4 * N, 32)

    s1o, s2o, s3o = _dgn_sc(xq2, src, dst, ea, ee)
    Xs = _finish_tc(s1o, s2o, s3o, scal, x, W_dgn, b_dgn,
                    theta1_W, theta1_b)
    Xflat = Xs.reshape(2 * N, 64)
    ivp, iep, ivn, ien = _inv_tc(cvp, cep, cvn, cen)

    xop, xon, _, _ = _hg_sc(True, Xflat, Xflat, pv, pe, nv, ne,
                            iep, ivp, ien, ivn)
    Xps, Xns = _theta2_tc(xop, xon, theta2_W, theta2_b)
    xop2, xon2, yp2, yn2 = _hg_sc(False, Xps.reshape(2 * N, 64),
                                  Xns.reshape(2 * N, 64), pv, pe, nv, ne,
                                  iep, ivp, ien, ivn)

    X1 = jnp.concatenate([xop2[:N], xop2[N:]], axis=-1)
    X2 = jnp.concatenate([xon2[:N], xon2[N:]], axis=-1)
    Yp = jnp.concatenate([yp2[:NHE], yp2[NHE:]], axis=-1)
    Yn = jnp.concatenate([yn2[:NHE], yn2[NHE:]], axis=-1)
    return (X1, X2, Yp, Yn)


# speedup vs baseline: 10.7518x; 1.7333x over previous
import jax
import jax.numpy as jnp
from jax import lax
from jax.experimental import pallas as pl
from jax.experimental.pallas import tpu as pltpu
from jax.experimental.pallas import tpu_sc as plsc

N = 10000
E = 320000
D_IN = 2816
D = 128
NHE = 2000
P = 40000
DELTA = 2.5

_SC_MESH = plsc.VectorSubcoreMesh(core_axis_name="c", subcore_axis_name="s")
_SC_PARAMS = pltpu.CompilerParams(use_tc_tiling_on_sc=False,
                                  needs_layout_passes=False)

_F32 = jnp.float32
_I32 = jnp.int32


def _lane_consts():
    io = lax.iota(_I32, 16)
    ones_row = jnp.where(io == 0, 1.0, 0.0).astype(_F32)
    deg_row = jnp.where(io == 1, 1.0, 0.0).astype(_F32)
    e2_row = jnp.where(io == 2, 1.0, 0.0).astype(_F32)
    return ones_row, deg_row, e2_row


def _bcast(buf, j):
    return plsc.load_gather(buf, [jnp.full((16,), j, _I32)])


def _zero_vmem(buf, rows, width):
    z = jnp.zeros((16,), _F32)

    @pl.loop(0, rows)
    def _(r):
        for f in range(0, width, 16):
            buf[r, pl.ds(f, 16)] = z


_MLP_BN = 400


def _mlp_body(m_ref, w1_ref, b1_ref, w2_ref, b2_ref, w3_ref, b3_ref,
              x_ref, xq_ref):
    h = jnp.maximum(
        jnp.dot(m_ref[...], w1_ref[...], preferred_element_type=_F32)
        + b1_ref[...], 0.0)
    h = jnp.maximum(
        jnp.dot(h, w2_ref[...], preferred_element_type=_F32)
        + b2_ref[...], 0.0)
    x = jnp.dot(h, w3_ref[...], preferred_element_type=_F32) + b3_ref[...]
    x_ref[...] = x
    xq_ref[...] = jnp.stack([x[:, 0:32], x[:, 32:64], x[:, 64:96], x[:, 96:128]])


def _mlp_tc(m_emb, W1, b1, W2, b2, W3, b3):
    grid = N // _MLP_BN
    return pl.pallas_call(
        _mlp_body,
        grid=(grid,),
        in_specs=[
            pl.BlockSpec((_MLP_BN, D_IN), lambda i: (i, 0)),
            pl.BlockSpec((D_IN, 512), lambda i: (0, 0)),
            pl.BlockSpec((512,), lambda i: (0,)),
            pl.BlockSpec((512, 512), lambda i: (0, 0)),
            pl.BlockSpec((512,), lambda i: (0,)),
            pl.BlockSpec((512, D), lambda i: (0, 0)),
            pl.BlockSpec((D,), lambda i: (0,)),
        ],
        out_specs=[
            pl.BlockSpec((_MLP_BN, D), lambda i: (i, 0)),
            pl.BlockSpec((4, _MLP_BN, 32), lambda i: (0, i, 0)),
        ],
        out_shape=[
            jax.ShapeDtypeStruct((N, D), _F32),
            jax.ShapeDtypeStruct((4, N, 32), _F32),
        ],
    )(m_emb, W1, b1, W2, b2, W3, b3)


_PK = 128
_PNC = E // _PK
_HK = 80
_HNC = P // _HK


def _prep_body(src_hbm, dst_hbm, eig2_hbm, pv_hbm, pe_hbm, nv_hbm, ne_hbm,
               ea_hbm, ee_hbm, scal_hbm, cvp_hbm, cep_hbm, cvn_hbm, cen_hbm,
               scal_sp, cvp_sp, cep_sp, cvn_sp, cen_sp,
               eig2_v, isrc, idst, abuf, ebuf, rowbuf, iv, ie, onesbuf,
               zb, sem):
    c = lax.axis_index("c")
    sid = lax.axis_index("s")
    g = c * 16 + sid
    ones_row, deg_row, e2_row = _lane_consts()

    _zero_vmem(zb, 125, 16)

    @pl.loop(0, 5)
    def _(k):
        rr = sid * 625 + k * 125
        pltpu.sync_copy(zb, scal_sp.at[pl.ds(rr, 125)])
        pltpu.sync_copy(zb, cvp_sp.at[pl.ds(rr, 125)])
        pltpu.sync_copy(zb, cvn_sp.at[pl.ds(rr, 125)])

    pltpu.sync_copy(zb, cep_sp.at[pl.ds(sid * 125, 125)])
    pltpu.sync_copy(zb, cen_sp.at[pl.ds(sid * 125, 125)])

    @pl.loop(0, _HK)
    def _(r):
        onesbuf[r, pl.ds(0, 16)] = ones_row

    pltpu.sync_copy(eig2_hbm, eig2_v)
    plsc.subcore_barrier()

    @pl.loop(g, _PNC, step=32)
    def _(ch):
        base = ch * _PK
        pltpu.sync_copy(src_hbm.at[pl.ds(base, _PK)], isrc)
        pltpu.sync_copy(dst_hbm.at[pl.ds(base, _PK)], idst)

        @pl.loop(0, _PK, step=16)
        def _(j):
            s16 = isrc[pl.ds(j, 16)]
            d16 = idst[pl.ds(j, 16)]
            ev = plsc.load_gather(eig2_v, [d16]) - plsc.load_gather(eig2_v, [s16])
            ebuf[pl.ds(j, 16)] = ev
            abuf[pl.ds(j, 16)] = jnp.abs(ev)

        @pl.loop(0, _PK)
        def _(j):
            fa = _bcast(abuf, j)
            fe = _bcast(ebuf, j)
            rowbuf[j, pl.ds(0, 16)] = ones_row + fa * deg_row + fe * e2_row

        pltpu.sync_copy(abuf, ea_hbm.at[pl.ds(base, _PK)])
        pltpu.sync_copy(ebuf, ee_hbm.at[pl.ds(base, _PK)])
        pltpu.sync_copy(rowbuf, scal_sp.at[idst], add=True)

    @pl.loop(g, _HNC, step=32)
    def _(ch):
        base = ch * _HK
        pltpu.sync_copy(pv_hbm.at[pl.ds(base, _HK)], iv)
        pltpu.sync_copy(pe_hbm.at[pl.ds(base, _HK)], ie)
        d1 = pltpu.async_copy(onesbuf, cvp_sp.at[iv], sem, add=True)
        d2 = pltpu.async_copy(onesbuf, cep_sp.at[ie], sem, add=True)
        d1.wait()
        d2.wait()
        pltpu.sync_copy(nv_hbm.at[pl.ds(base, _HK)], iv)
        pltpu.sync_copy(ne_hbm.at[pl.ds(base, _HK)], ie)
        d3 = pltpu.async_copy(onesbuf, cvn_sp.at[iv], sem, add=True)
        d4 = pltpu.async_copy(onesbuf, cen_sp.at[ie], sem, add=True)
        d3.wait()
        d4.wait()

    plsc.subcore_barrier()

    @pl.loop(0, 5)
    def _(k):
        rr = sid * 625 + k * 125
        orow = c * N + rr
        pltpu.sync_copy(scal_sp.at[pl.ds(rr, 125)], zb)
        pltpu.sync_copy(zb, scal_hbm.at[pl.ds(orow, 125)])
        pltpu.sync_copy(cvp_sp.at[pl.ds(rr, 125)], zb)
        pltpu.sync_copy(zb, cvp_hbm.at[pl.ds(orow, 125)])
        pltpu.sync_copy(cvn_sp.at[pl.ds(rr, 125)], zb)
        pltpu.sync_copy(zb, cvn_hbm.at[pl.ds(orow, 125)])

    re = sid * 125
    oe = c * NHE + re
    pltpu.sync_copy(cep_sp.at[pl.ds(re, 125)], zb)
    pltpu.sync_copy(zb, cep_hbm.at[pl.ds(oe, 125)])
    pltpu.sync_copy(cen_sp.at[pl.ds(re, 125)], zb)
    pltpu.sync_copy(zb, cen_hbm.at[pl.ds(oe, 125)])


def _prep_sc(src, dst, eig2, pv, pe, nv, ne):
    fn = pl.kernel(
        _prep_body,
        out_type=(
            jax.ShapeDtypeStruct((E,), _F32),
            jax.ShapeDtypeStruct((E,), _F32),
            jax.ShapeDtypeStruct((2 * N, 16), _F32),
            jax.ShapeDtypeStruct((2 * N, 16), _F32),
            jax.ShapeDtypeStruct((2 * NHE, 16), _F32),
            jax.ShapeDtypeStruct((2 * N, 16), _F32),
            jax.ShapeDtypeStruct((2 * NHE, 16), _F32),
        ),
        mesh=_SC_MESH,
        scratch_types=[
            pltpu.VMEM_SHARED((N, 16), _F32),
            pltpu.VMEM_SHARED((N, 16), _F32),
            pltpu.VMEM_SHARED((NHE, 16), _F32),
            pltpu.VMEM_SHARED((N, 16), _F32),
            pltpu.VMEM_SHARED((NHE, 16), _F32),
            pltpu.VMEM((N,), _F32),
            pltpu.VMEM((_PK,), _I32),
            pltpu.VMEM((_PK,), _I32),
            pltpu.VMEM((_PK,), _F32),
            pltpu.VMEM((_PK,), _F32),
            pltpu.VMEM((_PK, 16), _F32),
            pltpu.VMEM((_HK,), _I32),
            pltpu.VMEM((_HK,), _I32),
            pltpu.VMEM((_HK, 16), _F32),
            pltpu.VMEM((125, 16), _F32),
            pltpu.SemaphoreType.DMA,
        ],
        compiler_params=_SC_PARAMS,
    )
    return fn(src, dst, eig2, pv, pe, nv, ne)


_K = 128
_NCHUNK = E // _K
_CPT = 156
_RPS = N // 16
_RC = 125


def _dgn_sc_body(xq_hbm, src_hbm, dst_hbm, ea_hbm, ee_hbm,
                 s1_hbm, s23_hbm,
                 s1_sp, s23_sp,
                 isrc, idst, abuf, ebuf, igat, msg, mad, zb32, zb64,
                 sg0, sg1, sg2, sg3, sm0, sm1, sm2, sm3, ss0, ss1, ss2, ss3):
    c = lax.axis_index("c")
    sid = lax.axis_index("s")
    r0 = sid * _RPS
    sg = (sg0, sg1, sg2, sg3)
    sm = (sm0, sm1, sm2, sm3)
    ss = (ss0, ss1, ss2, ss3)

    def meta_issue(ch, st, sem):
        base = ch * _K
        d = [pltpu.async_copy(src_hbm.at[pl.ds(base, _K)], isrc.at[st], sem),
             pltpu.async_copy(dst_hbm.at[pl.ds(base, _K)], idst.at[st], sem),
             pltpu.async_copy(ea_hbm.at[pl.ds(base, _K)], abuf.at[st], sem),
             pltpu.async_copy(ee_hbm.at[pl.ds(base, _K)], ebuf.at[st], sem)]
        return d

    def meta_wait(ch, st, sem):
        base = ch * _K
        pltpu.make_async_copy(src_hbm.at[pl.ds(base, _K)], isrc.at[st], sem).wait()
        pltpu.make_async_copy(dst_hbm.at[pl.ds(base, _K)], idst.at[st], sem).wait()
        pltpu.make_async_copy(ea_hbm.at[pl.ds(base, _K)], abuf.at[st], sem).wait()
        pltpu.make_async_copy(ee_hbm.at[pl.ds(base, _K)], ebuf.at[st], sem).wait()

    def msg_sl(st):
        return msg.at[pl.ds(st * _K, _K)]

    def mad_sl(st):
        return mad.at[pl.ds(st * _K, _K)]

    def build_igat(st, coff):
        for j in range(0, _K, 16):
            igat[st, pl.ds(j, 16)] = isrc[st, pl.ds(j, 16)] + coff

    def gather_issue(st):
        pltpu.async_copy(xq_hbm.at[igat.at[st]], msg_sl(st), sg[st])

    def gather_wait(st):
        pltpu.make_async_copy(xq_hbm.at[igat.at[st]], msg_sl(st), sg[st]).wait()

    def compute(st):
        @pl.loop(0, _K)
        def _(j):
            jj = jnp.full((16,), j, _I32)
            cc = jnp.full((16,), st, _I32)
            fa = plsc.load_gather(abuf, [cc, jj])
            fe = plsc.load_gather(ebuf, [cc, jj])
            row = st * _K + j
            m0 = msg[row, pl.ds(0, 16)]
            m1 = msg[row, pl.ds(16, 16)]
            mad[row, pl.ds(0, 16)] = m0 * fa
            mad[row, pl.ds(16, 16)] = m1 * fa
            mad[row, pl.ds(32, 16)] = m0 * fe
            mad[row, pl.ds(48, 16)] = m1 * fe

    def scatter_issue(st):
        pltpu.async_copy(msg_sl(st), s1_sp.at[idst.at[st]], ss[st], add=True)
        pltpu.async_copy(mad_sl(st), s23_sp.at[idst.at[st]], ss[st], add=True)

    def scatter_wait(st):
        pltpu.make_async_copy(msg_sl(st), s1_sp.at[idst.at[st]], ss[st]).wait()
        pltpu.make_async_copy(mad_sl(st), s23_sp.at[idst.at[st]], ss[st]).wait()

    for p in range(2):
        q = 2 * c + p
        coff = q * N
        _zero_vmem(zb32, _RC, 32)
        _zero_vmem(zb64, _RC, 64)

        @pl.loop(0, 5)
        def _(k):
            rr = r0 + k * _RC
            pltpu.sync_copy(zb32, s1_sp.at[pl.ds(rr, _RC)])
            pltpu.sync_copy(zb64, s23_sp.at[pl.ds(rr, _RC)])

        plsc.subcore_barrier()

        c0 = sid * _CPT
        for d in meta_issue(c0, 0, sm[0]):
            d.wait()
        build_igat(0, coff)
        gather_issue(0)
        meta_issue(c0 + 1, 1, sm[1])

        @pl.loop(0, _CPT // 4)
        def _(kk):
            for ci in range(4):
                kloc = 4 * kk + ci
                ch = c0 + kloc

                @pl.when(kloc >= 2)
                def _():
                    scatter_wait((ci + 2) % 4)

                @pl.when(kloc + 2 < _CPT)
                def _():
                    meta_issue(ch + 2, (ci + 2) % 4, sm[(ci + 2) % 4])

                @pl.when(kloc + 1 < _CPT)
                def _():
                    meta_wait(ch + 1, (ci + 1) % 4, sm[(ci + 1) % 4])
                    build_igat((ci + 1) % 4, coff)
                    gather_issue((ci + 1) % 4)

                gather_wait(ci)
                compute(ci)
                scatter_issue(ci)

        scatter_wait(2)
        scatter_wait(3)

        @pl.when(sid < 4)
        def _():
            ch = 16 * _CPT + sid
            for d in meta_issue(ch, 0, sm[0]):
                d.wait()
            build_igat(0, coff)
            gather_issue(0)
            gather_wait(0)
            compute(0)
            scatter_issue(0)
            scatter_wait(0)

        plsc.subcore_barrier()

        @pl.loop(0, 5)
        def _(k):
            rr = r0 + k * _RC
            orow = coff + rr
            pltpu.sync_copy(s1_sp.at[pl.ds(rr, _RC)], zb32)
            pltpu.sync_copy(zb32, s1_hbm.at[pl.ds(orow, _RC)])
            pltpu.sync_copy(s23_sp.at[pl.ds(rr, _RC)], zb64)
            pltpu.sync_copy(zb64, s23_hbm.at[pl.ds(orow, _RC)])

        plsc.subcore_barrier()


def _dgn_sc(xq, src, dst, ea, ee):
    fn = pl.kernel(
        _dgn_sc_body,
        out_type=(
            jax.ShapeDtypeStruct((4 * N, 32), _F32),
            jax.ShapeDtypeStruct((4 * N, 64), _F32),
        ),
        mesh=_SC_MESH,
        scratch_types=[
            pltpu.VMEM_SHARED((N, 32), _F32),
            pltpu.VMEM_SHARED((N, 64), _F32),
            pltpu.VMEM((4, _K), _I32),
            pltpu.VMEM((4, _K), _I32),
            pltpu.VMEM((4, _K), _F32),
            pltpu.VMEM((4, _K), _F32),
            pltpu.VMEM((4, _K), _I32),
            pltpu.VMEM((4 * _K, 32), _F32),
            pltpu.VMEM((4 * _K, 64), _F32),
            pltpu.VMEM((_RC, 32), _F32),
            pltpu.VMEM((_RC, 64), _F32),
        ] + [pltpu.SemaphoreType.DMA] * 12,
        compiler_params=_SC_PARAMS,
    )
    return fn(xq, src, dst, ea, ee)


def _inv_body(cvp_ref, cep_ref, cvn_ref, cen_ref,
              ivp_ref, iep_ref, ivn_ref, ien_ref):
    def inv(ref, n):
        a = ref[...]
        r = 1.0 / jnp.maximum(a[:n, 0] + a[n:, 0], 1.0)
        return jnp.broadcast_to(r[:, None], (n, 16))

    ivp_ref[...] = inv(cvp_ref, N)
    iep_ref[...] = inv(cep_ref, NHE)
    ivn_ref[...] = inv(cvn_ref, N)
    ien_ref[...] = inv(cen_ref, NHE)


def _inv_tc(cvp, cep, cvn, cen):
    return pl.pallas_call(
        _inv_body,
        out_shape=[
            jax.ShapeDtypeStruct((N, 16), _F32),
            jax.ShapeDtypeStruct((NHE, 16), _F32),
            jax.ShapeDtypeStruct((N, 16), _F32),
            jax.ShapeDtypeStruct((NHE, 16), _F32),
        ],
    )(cvp, cep, cvn, cen)


_FB = 400


def _finish_body(s1q0, s1q1, s1q2, s1q3, sq0, sq1, sq2, sq3, sca, scb,
                 x_ref, wd_ref, bd_ref, t1w_ref, t1b_ref, xs_ref):
    s1 = jnp.concatenate(
        [s1q0[...], s1q1[...], s1q2[...], s1q3[...]], axis=-1)
    s2 = jnp.concatenate(
        [sq0[...][:, :32], sq1[...][:, :32], sq2[...][:, :32],
         sq3[...][:, :32]], axis=-1)
    s3 = jnp.concatenate(
        [sq0[...][:, 32:], sq1[...][:, 32:], sq2[...][:, 32:],
         sq3[...][:, 32:]], axis=-1)
    sc = sca[...] + scb[...]
    deg = sc[:, 0]
    s4 = sc[:, 1]
    s5 = sc[:, 2]
    x = x_ref[...]
    inv = 1.0 / (s4 + 1e-8)
    agg_av = s2 * inv[:, None]
    agg_dx = s3 * inv[:, None] - x * (s5 * inv)[:, None]
    amp = jnp.log(deg + 1.0) * (1.0 / DELTA)
    h = jnp.concatenate([agg_av, agg_dx, s1], axis=-1)
    h6 = jnp.concatenate([h, h * amp[:, None]], axis=-1)
    xd = jnp.dot(h6, wd_ref[...], preferred_element_type=_F32) + bd_ref[...]
    X = jnp.dot(xd, t1w_ref[...], preferred_element_type=_F32) + t1b_ref[...]
    xs_ref[...] = jnp.stack([X[:, :64], X[:, 64:]])


def _finish_tc(s1o, s23o, scal, x, W_dgn, b_dgn, t1w, t1b):
    grid = N // _FB
    off = N // _FB

    def qspec(q, w):
        return pl.BlockSpec((_FB, w), lambda i, q=q: (i + q * off, 0))

    return pl.pallas_call(
        _finish_body,
        grid=(grid,),
        in_specs=[
            qspec(0, 32), qspec(1, 32), qspec(2, 32), qspec(3, 32),
            qspec(0, 64), qspec(1, 64), qspec(2, 64), qspec(3, 64),
            pl.BlockSpec((_FB, 16), lambda i: (i, 0)),
            pl.BlockSpec((_FB, 16), lambda i: (i + off, 0)),
            pl.BlockSpec((_FB, D), lambda i: (i, 0)),
            pl.BlockSpec((6 * D, D), lambda i: (0, 0)),
            pl.BlockSpec((D,), lambda i: (0,)),
            pl.BlockSpec((D, D), lambda i: (0, 0)),
            pl.BlockSpec((D,), lambda i: (0,)),
        ],
        out_specs=pl.BlockSpec((2, _FB, 64), lambda i: (0, i, 0)),
        out_shape=jax.ShapeDtypeStruct((2, N, 64), _F32),
    )(s1o, s1o, s1o, s1o, s23o, s23o, s23o, s23o,
      scal, scal, x, W_dgn, b_dgn, t1w, t1b)


_YRPS = NHE // 16
_XRPS = N // 16


def _hg_body(relu,
             xp_hbm, xn_hbm, pv_hbm, pe_hbm, nv_hbm, ne_hbm,
             iep_hbm, ivp_hbm, ien_hbm, ivn_hbm,
             xop_hbm, xon_hbm, yp_hbm, yn_hbm,
             yp_sp, yn_sp, xop_sp, xon_sp,
             iv, ie, igat, buf, invb, zb, sem):
    c = lax.axis_index("c")
    sid = lax.axis_index("s")
    _zero_vmem(zb, _RC, 64)
    ry = sid * _YRPS
    rx = sid * _XRPS

    pltpu.sync_copy(zb.at[pl.ds(0, _YRPS)], yp_sp.at[pl.ds(ry, _YRPS)])
    pltpu.sync_copy(zb.at[pl.ds(0, _YRPS)], yn_sp.at[pl.ds(ry, _YRPS)])

    @pl.loop(0, 5)
    def _(k):
        rr = rx + k * _RC
        pltpu.sync_copy(zb, xop_sp.at[pl.ds(rr, _RC)])
        pltpu.sync_copy(zb, xon_sp.at[pl.ds(rr, _RC)])

    plsc.subcore_barrier()

    coff = c * N

    for (v_hbm, e_hbm, x_hbm, y_sp) in (
            (pv_hbm, pe_hbm, xp_hbm, yp_sp),
            (nv_hbm, ne_hbm, xn_hbm, yn_sp)):
        @pl.loop(sid, _HNC, step=16)
        def _(ch):
            base = ch * _HK
            pltpu.sync_copy(v_hbm.at[pl.ds(base, _HK)], iv)
            pltpu.sync_copy(e_hbm.at[pl.ds(base, _HK)], ie)

            @pl.loop(0, _HK, step=16)
            def _(j):
                igat[pl.ds(j, 16)] = iv[pl.ds(j, 16)] + coff

            pltpu.async_copy(x_hbm.at[igat], buf, sem).wait()
            pltpu.sync_copy(buf, y_sp.at[ie], add=True)

    plsc.subcore_barrier()

    for (ie_hbm, y_sp, y_hbm) in ((iep_hbm, yp_sp, yp_hbm),
                                  (ien_hbm, yn_sp, yn_hbm)):
        pltpu.sync_copy(y_sp.at[pl.ds(ry, _YRPS)], zb.at[pl.ds(0, _YRPS)])
        pltpu.sync_copy(ie_hbm.at[pl.ds(ry, _YRPS)], invb)

        @pl.loop(0, _YRPS)
        def _(r):
            fi = invb[r, pl.ds(0, 16)]
            for f in range(4):
                zb[r, pl.ds(f * 16, 16)] = zb[r, pl.ds(f * 16, 16)] * fi

        pltpu.sync_copy(zb.at[pl.ds(0, _YRPS)], y_hbm.at[pl.ds(c * NHE + ry, _YRPS)])

    plsc.subcore_barrier()

    yoff = c * NHE
    for (v_hbm, e_hbm, y_hbm, xo_sp) in (
            (pv_hbm, pe_hbm, yp_hbm, xop_sp),
            (nv_hbm, ne_hbm, yn_hbm, xon_sp)):
        @pl.loop(sid, _HNC, step=16)
        def _(ch):
            base = ch * _HK
            pltpu.sync_copy(v_hbm.at[pl.ds(base, _HK)], iv)
            pltpu.sync_copy(e_hbm.at[pl.ds(base, _HK)], ie)

            @pl.loop(0, _HK, step=16)
            def _(j):
                igat[pl.ds(j, 16)] = ie[pl.ds(j, 16)] + yoff

            pltpu.async_copy(y_hbm.at[igat], buf, sem).wait()
            pltpu.sync_copy(buf, xo_sp.at[iv], add=True)

    plsc.subcore_barrier()

    for (iv_hbm, xo_sp, xo_hbm) in ((ivp_hbm, xop_sp, xop_hbm),
                                    (ivn_hbm, xon_sp, xon_hbm)):
        @pl.loop(0, 5)
        def _(k):
            rr = rx + k * _RC
            pltpu.sync_copy(xo_sp.at[pl.ds(rr, _RC)], zb)
            pltpu.sync_copy(iv_hbm.at[pl.ds(rr, _RC)], invb)

            @pl.loop(0, _RC)
            def _(r):
                fi = invb[r, pl.ds(0, 16)]
                for f in range(4):
                    val = zb[r, pl.ds(f * 16, 16)] * fi
                    if relu:
                        val = jnp.maximum(val, 0.0)
                    zb[r, pl.ds(f * 16, 16)] = val

            pltpu.sync_copy(zb, xo_hbm.at[pl.ds(coff + rr, _RC)])


def _hg_sc(relu, xp, xn, pv, pe, nv, ne, iep, ivp, ien, ivn):
    def body(*refs):
        return _hg_body(relu, *refs)

    fn = pl.kernel(
        body,
        out_type=(
            jax.ShapeDtypeStruct((2 * N, 64), _F32),
            jax.ShapeDtypeStruct((2 * N, 64), _F32),
            jax.ShapeDtypeStruct((2 * NHE, 64), _F32),
            jax.ShapeDtypeStruct((2 * NHE, 64), _F32),
        ),
        mesh=_SC_MESH,
        scratch_types=[
            pltpu.VMEM_SHARED((NHE, 64), _F32),
            pltpu.VMEM_SHARED((NHE, 64), _F32),
            pltpu.VMEM_SHARED((N, 64), _F32),
            pltpu.VMEM_SHARED((N, 64), _F32),
            pltpu.VMEM((_HK,), _I32),
            pltpu.VMEM((_HK,), _I32),
            pltpu.VMEM((_HK,), _I32),
            pltpu.VMEM((_HK, 64), _F32),
            pltpu.VMEM((_RC, 16), _F32),
            pltpu.VMEM((_RC, 64), _F32),
            pltpu.SemaphoreType.DMA,
        ],
        compiler_params=_SC_PARAMS,
    )
    return fn(xp, xn, pv, pe, nv, ne, iep, ivp, ien, ivn)


def _theta2_body(xpa, xpb, xna, xnb, w_ref, b_ref, op_ref, on_ref):
    w = w_ref[...]
    b = b_ref[...]
    Xp = jnp.concatenate([xpa[...], xpb[...]], axis=-1)
    Xn = jnp.concatenate([xna[...], xnb[...]], axis=-1)
    Yp = jnp.dot(Xp, w, preferred_element_type=_F32) + b
    Yn = jnp.dot(Xn, w, preferred_element_type=_F32) + b
    op_ref[...] = jnp.stack([Yp[:, :64], Yp[:, 64:]])
    on_ref[...] = jnp.stack([Yn[:, :64], Yn[:, 64:]])


def _theta2_tc(xop, xon, t2w, t2b):
    grid = N // _FB
    off = N // _FB
    return pl.pallas_call(
        _theta2_body,
        grid=(grid,),
        in_specs=[
            pl.BlockSpec((_FB, 64), lambda i: (i, 0)),
            pl.BlockSpec((_FB, 64), lambda i: (i + off, 0)),
            pl.BlockSpec((_FB, 64), lambda i: (i, 0)),
            pl.BlockSpec((_FB, 64), lambda i: (i + off, 0)),
            pl.BlockSpec((D, D), lambda i: (0, 0)),
            pl.BlockSpec((D,), lambda i: (0,)),
        ],
        out_specs=[
            pl.BlockSpec((2, _FB, 64), lambda i: (0, i, 0)),
            pl.BlockSpec((2, _FB, 64), lambda i: (0, i, 0)),
        ],
        out_shape=[
            jax.ShapeDtypeStruct((2, N, 64), _F32),
            jax.ShapeDtypeStruct((2, N, 64), _F32),
        ],
    )(xop, xop, xon, xon, t2w, t2b)


def kernel(m_emb, edge_index, eig, hg_pos_v, hg_pos_e, hg_neg_v, hg_neg_e,
           W1, b1, W2, b2, W3, b3, W_dgn, b_dgn,
           theta1_W, theta1_b, theta2_W, theta2_b):
    src = edge_index[0].astype(_I32)
    dst = edge_index[1].astype(_I32)
    eig2 = jnp.asarray(eig[:, 2], _F32)
    pv = hg_pos_v.astype(_I32)
    pe = hg_pos_e.astype(_I32)
    nv = hg_neg_v.astype(_I32)
    ne = hg_neg_e.astype(_I32)

    ea, ee, scal, cvp, cep, cvn, cen = _prep_sc(src, dst, eig2, pv, pe, nv, ne)
    x, xq = _mlp_tc(m_emb, W1, b1, W2, b2, W3, b3)
    xq2 = xq.reshape(4 * N, 32)

    s1o, s23o = _dgn_sc(xq2, src, dst, ea, ee)
    Xs = _finish_tc(s1o, s23o, scal, x, W_dgn, b_dgn,
                    theta1_W, theta1_b)
    Xflat = Xs.reshape(2 * N, 64)
    ivp, iep, ivn, ien = _inv_tc(cvp, cep, cvn, cen)

    xop, xon, _, _ = _hg_sc(True, Xflat, Xflat, pv, pe, nv, ne,
                            iep, ivp, ien, ivn)
    Xps, Xns = _theta2_tc(xop, xon, theta2_W, theta2_b)
    xop2, xon2, yp2, yn2 = _hg_sc(False, Xps.reshape(2 * N, 64),
                                  Xns.reshape(2 * N, 64), pv, pe, nv, ne,
                                  iep, ivp, ien, ivn)

    X1 = jnp.concatenate([xop2[:N], xop2[N:]], axis=-1)
    X2 = jnp.concatenate([xon2[:N], xon2[N:]], axis=-1)
    Yp = jnp.concatenate([yp2[:NHE], yp2[NHE:]], axis=-1)
    Yn = jnp.concatenate([yn2[:NHE], yn2[NHE:]], axis=-1)
    return (X1, X2, Yp, Yn)


# speedup vs baseline: 14.8960x; 1.3854x over previous
import jax
import jax.numpy as jnp
from jax import lax
from jax.experimental import pallas as pl
from jax.experimental.pallas import tpu as pltpu
from jax.experimental.pallas import tpu_sc as plsc

N = 10000
E = 320000
D_IN = 2816
D = 128
NHE = 2000
P = 40000
DELTA = 2.5

_SC_MESH = plsc.VectorSubcoreMesh(core_axis_name="c", subcore_axis_name="s")
_SC_PARAMS = pltpu.CompilerParams(use_tc_tiling_on_sc=False,
                                  needs_layout_passes=False)

_F32 = jnp.float32
_I32 = jnp.int32


def _lane_consts():
    io = lax.iota(_I32, 16)
    ones_row = jnp.where(io == 0, 1.0, 0.0).astype(_F32)
    deg_row = jnp.where(io == 1, 1.0, 0.0).astype(_F32)
    e2_row = jnp.where(io == 2, 1.0, 0.0).astype(_F32)
    return ones_row, deg_row, e2_row


def _bcast(buf, j):
    return plsc.load_gather(buf, [jnp.full((16,), j, _I32)])


def _zero_vmem(buf, rows, width):
    z = jnp.zeros((16,), _F32)

    @pl.loop(0, rows)
    def _(r):
        for f in range(0, width, 16):
            buf[r, pl.ds(f, 16)] = z


_MLP_BN = 400


def _mlp_body(m_ref, w1_ref, b1_ref, w2_ref, b2_ref, w3_ref, b3_ref,
              x_ref, xq_ref):
    h = jnp.maximum(
        jnp.dot(m_ref[...], w1_ref[...], preferred_element_type=_F32)
        + b1_ref[...], 0.0)
    h = jnp.maximum(
        jnp.dot(h, w2_ref[...], preferred_element_type=_F32)
        + b2_ref[...], 0.0)
    x = jnp.dot(h, w3_ref[...], preferred_element_type=_F32) + b3_ref[...]
    x_ref[...] = x
    xq_ref[...] = jnp.stack([x[:, 0:32], x[:, 32:64], x[:, 64:96], x[:, 96:128]])


def _mlp_tc(m_emb, W1, b1, W2, b2, W3, b3):
    grid = N // _MLP_BN
    return pl.pallas_call(
        _mlp_body,
        grid=(grid,),
        in_specs=[
            pl.BlockSpec((_MLP_BN, D_IN), lambda i: (i, 0)),
            pl.BlockSpec((D_IN, 512), lambda i: (0, 0)),
            pl.BlockSpec((512,), lambda i: (0,)),
            pl.BlockSpec((512, 512), lambda i: (0, 0)),
            pl.BlockSpec((512,), lambda i: (0,)),
            pl.BlockSpec((512, D), lambda i: (0, 0)),
            pl.BlockSpec((D,), lambda i: (0,)),
        ],
        out_specs=[
            pl.BlockSpec((_MLP_BN, D), lambda i: (i, 0)),
            pl.BlockSpec((4, _MLP_BN, 32), lambda i: (0, i, 0)),
        ],
        out_shape=[
            jax.ShapeDtypeStruct((N, D), _F32),
            jax.ShapeDtypeStruct((4, N, 32), _F32),
        ],
    )(m_emb, W1, b1, W2, b2, W3, b3)


_PK = 128
_PNC = E // _PK
_HK = 80
_HNC = P // _HK


def _prep_body(src_hbm, dst_hbm, eig2_hbm, pv_hbm, pe_hbm, nv_hbm, ne_hbm,
               ea_hbm, ee_hbm, scal_hbm, cvp_hbm, cep_hbm, cvn_hbm, cen_hbm,
               scal_sp, cvp_sp, cep_sp, cvn_sp, cen_sp,
               eig2_v, isrc, idst, abuf, ebuf, rowbuf, iv, ie, onesbuf,
               zb, sem):
    c = lax.axis_index("c")
    sid = lax.axis_index("s")
    g = c * 16 + sid
    ones_row, deg_row, e2_row = _lane_consts()

    _zero_vmem(zb, 125, 16)

    @pl.loop(0, 5)
    def _(k):
        rr = sid * 625 + k * 125
        pltpu.sync_copy(zb, scal_sp.at[pl.ds(rr, 125)])
        pltpu.sync_copy(zb, cvp_sp.at[pl.ds(rr, 125)])
        pltpu.sync_copy(zb, cvn_sp.at[pl.ds(rr, 125)])

    pltpu.sync_copy(zb, cep_sp.at[pl.ds(sid * 125, 125)])
    pltpu.sync_copy(zb, cen_sp.at[pl.ds(sid * 125, 125)])

    @pl.loop(0, _HK)
    def _(r):
        onesbuf[r, pl.ds(0, 16)] = ones_row

    pltpu.sync_copy(eig2_hbm, eig2_v)
    plsc.subcore_barrier()

    @pl.loop(g, _PNC, step=32)
    def _(ch):
        base = ch * _PK
        pltpu.sync_copy(src_hbm.at[pl.ds(base, _PK)], isrc)
        pltpu.sync_copy(dst_hbm.at[pl.ds(base, _PK)], idst)

        @pl.loop(0, _PK, step=16)
        def _(j):
            s16 = isrc[pl.ds(j, 16)]
            d16 = idst[pl.ds(j, 16)]
            ev = plsc.load_gather(eig2_v, [d16]) - plsc.load_gather(eig2_v, [s16])
            ebuf[pl.ds(j, 16)] = ev
            abuf[pl.ds(j, 16)] = jnp.abs(ev)

        @pl.loop(0, _PK)
        def _(j):
            fa = _bcast(abuf, j)
            fe = _bcast(ebuf, j)
            rowbuf[j, pl.ds(0, 16)] = ones_row + fa * deg_row + fe * e2_row

        pltpu.sync_copy(abuf, ea_hbm.at[pl.ds(base, _PK)])
        pltpu.sync_copy(ebuf, ee_hbm.at[pl.ds(base, _PK)])
        pltpu.sync_copy(rowbuf, scal_sp.at[idst], add=True)

    @pl.loop(g, _HNC, step=32)
    def _(ch):
        base = ch * _HK
        pltpu.sync_copy(pv_hbm.at[pl.ds(base, _HK)], iv)
        pltpu.sync_copy(pe_hbm.at[pl.ds(base, _HK)], ie)
        d1 = pltpu.async_copy(onesbuf, cvp_sp.at[iv], sem, add=True)
        d2 = pltpu.async_copy(onesbuf, cep_sp.at[ie], sem, add=True)
        d1.wait()
        d2.wait()
        pltpu.sync_copy(nv_hbm.at[pl.ds(base, _HK)], iv)
        pltpu.sync_copy(ne_hbm.at[pl.ds(base, _HK)], ie)
        d3 = pltpu.async_copy(onesbuf, cvn_sp.at[iv], sem, add=True)
        d4 = pltpu.async_copy(onesbuf, cen_sp.at[ie], sem, add=True)
        d3.wait()
        d4.wait()

    plsc.subcore_barrier()

    @pl.loop(0, 5)
    def _(k):
        rr = sid * 625 + k * 125
        orow = c * N + rr
        pltpu.sync_copy(scal_sp.at[pl.ds(rr, 125)], zb)
        pltpu.sync_copy(zb, scal_hbm.at[pl.ds(orow, 125)])
        pltpu.sync_copy(cvp_sp.at[pl.ds(rr, 125)], zb)
        pltpu.sync_copy(zb, cvp_hbm.at[pl.ds(orow, 125)])
        pltpu.sync_copy(cvn_sp.at[pl.ds(rr, 125)], zb)
        pltpu.sync_copy(zb, cvn_hbm.at[pl.ds(orow, 125)])

    re = sid * 125
    oe = c * NHE + re
    pltpu.sync_copy(cep_sp.at[pl.ds(re, 125)], zb)
    pltpu.sync_copy(zb, cep_hbm.at[pl.ds(oe, 125)])
    pltpu.sync_copy(cen_sp.at[pl.ds(re, 125)], zb)
    pltpu.sync_copy(zb, cen_hbm.at[pl.ds(oe, 125)])


def _prep_sc(src, dst, eig2, pv, pe, nv, ne):
    fn = pl.kernel(
        _prep_body,
        out_type=(
            jax.ShapeDtypeStruct((E,), _F32),
            jax.ShapeDtypeStruct((E,), _F32),
            jax.ShapeDtypeStruct((2 * N, 16), _F32),
            jax.ShapeDtypeStruct((2 * N, 16), _F32),
            jax.ShapeDtypeStruct((2 * NHE, 16), _F32),
            jax.ShapeDtypeStruct((2 * N, 16), _F32),
            jax.ShapeDtypeStruct((2 * NHE, 16), _F32),
        ),
        mesh=_SC_MESH,
        scratch_types=[
            pltpu.VMEM_SHARED((N, 16), _F32),
            pltpu.VMEM_SHARED((N, 16), _F32),
            pltpu.VMEM_SHARED((NHE, 16), _F32),
            pltpu.VMEM_SHARED((N, 16), _F32),
            pltpu.VMEM_SHARED((NHE, 16), _F32),
            pltpu.VMEM((N,), _F32),
            pltpu.VMEM((_PK,), _I32),
            pltpu.VMEM((_PK,), _I32),
            pltpu.VMEM((_PK,), _F32),
            pltpu.VMEM((_PK,), _F32),
            pltpu.VMEM((_PK, 16), _F32),
            pltpu.VMEM((_HK,), _I32),
            pltpu.VMEM((_HK,), _I32),
            pltpu.VMEM((_HK, 16), _F32),
            pltpu.VMEM((125, 16), _F32),
            pltpu.SemaphoreType.DMA,
        ],
        compiler_params=_SC_PARAMS,
    )
    return fn(src, dst, eig2, pv, pe, nv, ne)


_K = 128
_NCHUNK = E // _K
_CPT = 156
_RPS = N // 16
_RC = 125


def _dgn_sc_body(xq_hbm, src_hbm, dst_hbm, ea_hbm, ee_hbm,
                 s1_hbm, s23_hbm,
                 s1_sp, s23_sp,
                 isrc, idst, abuf, ebuf, igat, msg, mad, zb32, zb64,
                 sg0, sg1, sg2, sg3, sm0, sm1, sm2, sm3, ss0, ss1, ss2, ss3):
    c = lax.axis_index("c")
    sid = lax.axis_index("s")
    r0 = sid * _RPS
    sg = (sg0, sg1, sg2, sg3)
    sm = (sm0, sm1, sm2, sm3)
    ss = (ss0, ss1, ss2, ss3)

    def meta_issue(ch, st, sem):
        base = ch * _K
        d = [pltpu.async_copy(src_hbm.at[pl.ds(base, _K)], isrc.at[st], sem),
             pltpu.async_copy(dst_hbm.at[pl.ds(base, _K)], idst.at[st], sem),
             pltpu.async_copy(ea_hbm.at[pl.ds(base, _K)], abuf.at[st], sem),
             pltpu.async_copy(ee_hbm.at[pl.ds(base, _K)], ebuf.at[st], sem)]
        return d

    def meta_wait(ch, st, sem):
        base = ch * _K
        pltpu.make_async_copy(src_hbm.at[pl.ds(base, _K)], isrc.at[st], sem).wait()
        pltpu.make_async_copy(dst_hbm.at[pl.ds(base, _K)], idst.at[st], sem).wait()
        pltpu.make_async_copy(ea_hbm.at[pl.ds(base, _K)], abuf.at[st], sem).wait()
        pltpu.make_async_copy(ee_hbm.at[pl.ds(base, _K)], ebuf.at[st], sem).wait()

    def msg_sl(st):
        return msg.at[pl.ds(st * _K, _K)]

    def mad_sl(st):
        return mad.at[pl.ds(st * _K, _K)]

    def build_igat(st, coff):
        for j in range(0, _K, 16):
            igat[st, pl.ds(j, 16)] = isrc[st, pl.ds(j, 16)] + coff

    def gather_issue(st):
        pltpu.async_copy(xq_hbm.at[igat.at[st]], msg_sl(st), sg[st])

    def gather_wait(st):
        pltpu.make_async_copy(xq_hbm.at[igat.at[st]], msg_sl(st), sg[st]).wait()

    def compute(st):
        @pl.loop(0, _K)
        def _(j):
            jj = jnp.full((16,), j, _I32)
            cc = jnp.full((16,), st, _I32)
            fa = plsc.load_gather(abuf, [cc, jj])
            fe = plsc.load_gather(ebuf, [cc, jj])
            row = st * _K + j
            m0 = msg[row, pl.ds(0, 16)]
            m1 = msg[row, pl.ds(16, 16)]
            mad[row, pl.ds(0, 16)] = m0 * fa
            mad[row, pl.ds(16, 16)] = m1 * fa
            mad[row, pl.ds(32, 16)] = m0 * fe
            mad[row, pl.ds(48, 16)] = m1 * fe

    def scatter_issue(st):
        pltpu.async_copy(msg_sl(st), s1_sp.at[idst.at[st]], ss[st], add=True)
        pltpu.async_copy(mad_sl(st), s23_sp.at[idst.at[st]], ss[st], add=True)

    def scatter_wait(st):
        pltpu.make_async_copy(msg_sl(st), s1_sp.at[idst.at[st]], ss[st]).wait()
        pltpu.make_async_copy(mad_sl(st), s23_sp.at[idst.at[st]], ss[st]).wait()

    for p in range(2):
        q = 2 * c + p
        coff = q * N
        _zero_vmem(zb32, _RC, 32)
        _zero_vmem(zb64, _RC, 64)

        @pl.loop(0, 5)
        def _(k):
            rr = r0 + k * _RC
            pltpu.sync_copy(zb32, s1_sp.at[pl.ds(rr, _RC)])
            pltpu.sync_copy(zb64, s23_sp.at[pl.ds(rr, _RC)])

        plsc.subcore_barrier()

        c0 = sid * _CPT
        for d in meta_issue(c0, 0, sm[0]):
            d.wait()
        build_igat(0, coff)
        gather_issue(0)
        meta_issue(c0 + 1, 1, sm[1])

        @pl.loop(0, _CPT // 4)
        def _(kk):
            for ci in range(4):
                kloc = 4 * kk + ci
                ch = c0 + kloc

                @pl.when(kloc >= 2)
                def _():
                    scatter_wait((ci + 2) % 4)

                @pl.when(kloc + 2 < _CPT)
                def _():
                    meta_issue(ch + 2, (ci + 2) % 4, sm[(ci + 2) % 4])

                @pl.when(kloc + 1 < _CPT)
                def _():
                    meta_wait(ch + 1, (ci + 1) % 4, sm[(ci + 1) % 4])
                    build_igat((ci + 1) % 4, coff)
                    gather_issue((ci + 1) % 4)

                gather_wait(ci)
                compute(ci)
                scatter_issue(ci)

        scatter_wait(2)
        scatter_wait(3)

        @pl.when(sid < 4)
        def _():
            ch = 16 * _CPT + sid
            for d in meta_issue(ch, 0, sm[0]):
                d.wait()
            build_igat(0, coff)
            gather_issue(0)
            gather_wait(0)
            compute(0)
            scatter_issue(0)
            scatter_wait(0)

        plsc.subcore_barrier()

        @pl.loop(0, 5)
        def _(k):
            rr = r0 + k * _RC
            orow = coff + rr
            pltpu.sync_copy(s1_sp.at[pl.ds(rr, _RC)], zb32)
            pltpu.sync_copy(zb32, s1_hbm.at[pl.ds(orow, _RC)])
            pltpu.sync_copy(s23_sp.at[pl.ds(rr, _RC)], zb64)
            pltpu.sync_copy(zb64, s23_hbm.at[pl.ds(orow, _RC)])

        plsc.subcore_barrier()


def _dgn_sc(xq, src, dst, ea, ee):
    fn = pl.kernel(
        _dgn_sc_body,
        out_type=(
            jax.ShapeDtypeStruct((4 * N, 32), _F32),
            jax.ShapeDtypeStruct((4 * N, 64), _F32),
        ),
        mesh=_SC_MESH,
        scratch_types=[
            pltpu.VMEM_SHARED((N, 32), _F32),
            pltpu.VMEM_SHARED((N, 64), _F32),
            pltpu.VMEM((4, _K), _I32),
            pltpu.VMEM((4, _K), _I32),
            pltpu.VMEM((4, _K), _F32),
            pltpu.VMEM((4, _K), _F32),
            pltpu.VMEM((4, _K), _I32),
            pltpu.VMEM((4 * _K, 32), _F32),
            pltpu.VMEM((4 * _K, 64), _F32),
            pltpu.VMEM((_RC, 32), _F32),
            pltpu.VMEM((_RC, 64), _F32),
        ] + [pltpu.SemaphoreType.DMA] * 12,
        compiler_params=_SC_PARAMS,
    )
    return fn(xq, src, dst, ea, ee)


def _inv_body(cvp_ref, cep_ref, cvn_ref, cen_ref,
              ivp_ref, iep_ref, ivn_ref, ien_ref):
    def inv(ref, n):
        a = ref[...]
        r = 1.0 / jnp.maximum(a[:n, 0] + a[n:, 0], 1.0)
        return jnp.broadcast_to(r[:, None], (n, 16))

    ivp_ref[...] = inv(cvp_ref, N)
    iep_ref[...] = inv(cep_ref, NHE)
    ivn_ref[...] = inv(cvn_ref, N)
    ien_ref[...] = inv(cen_ref, NHE)


def _inv_tc(cvp, cep, cvn, cen):
    return pl.pallas_call(
        _inv_body,
        out_shape=[
            jax.ShapeDtypeStruct((N, 16), _F32),
            jax.ShapeDtypeStruct((NHE, 16), _F32),
            jax.ShapeDtypeStruct((N, 16), _F32),
            jax.ShapeDtypeStruct((NHE, 16), _F32),
        ],
    )(cvp, cep, cvn, cen)


_FB = 400


def _finish_body(s1q0, s1q1, s1q2, s1q3, sq0, sq1, sq2, sq3, sca, scb,
                 x_ref, wd_ref, bd_ref, t1w_ref, t1b_ref, xs_ref):
    s1 = jnp.concatenate(
        [s1q0[...], s1q1[...], s1q2[...], s1q3[...]], axis=-1)
    s2 = jnp.concatenate(
        [sq0[...][:, :32], sq1[...][:, :32], sq2[...][:, :32],
         sq3[...][:, :32]], axis=-1)
    s3 = jnp.concatenate(
        [sq0[...][:, 32:], sq1[...][:, 32:], sq2[...][:, 32:],
         sq3[...][:, 32:]], axis=-1)
    sc = sca[...] + scb[...]
    deg = sc[:, 0]
    s4 = sc[:, 1]
    s5 = sc[:, 2]
    x = x_ref[...]
    inv = 1.0 / (s4 + 1e-8)
    agg_av = s2 * inv[:, None]
    agg_dx = s3 * inv[:, None] - x * (s5 * inv)[:, None]
    amp = jnp.log(deg + 1.0) * (1.0 / DELTA)
    h = jnp.concatenate([agg_av, agg_dx, s1], axis=-1)
    h6 = jnp.concatenate([h, h * amp[:, None]], axis=-1)
    xd = jnp.dot(h6, wd_ref[...], preferred_element_type=_F32) + bd_ref[...]
    X = jnp.dot(xd, t1w_ref[...], preferred_element_type=_F32) + t1b_ref[...]
    xs_ref[...] = jnp.stack([X[:, :64], X[:, 64:]])


def _finish_tc(s1o, s23o, scal, x, W_dgn, b_dgn, t1w, t1b):
    grid = N // _FB
    off = N // _FB

    def qspec(q, w):
        return pl.BlockSpec((_FB, w), lambda i, q=q: (i + q * off, 0))

    return pl.pallas_call(
        _finish_body,
        grid=(grid,),
        in_specs=[
            qspec(0, 32), qspec(1, 32), qspec(2, 32), qspec(3, 32),
            qspec(0, 64), qspec(1, 64), qspec(2, 64), qspec(3, 64),
            pl.BlockSpec((_FB, 16), lambda i: (i, 0)),
            pl.BlockSpec((_FB, 16), lambda i: (i + off, 0)),
            pl.BlockSpec((_FB, D), lambda i: (i, 0)),
            pl.BlockSpec((6 * D, D), lambda i: (0, 0)),
            pl.BlockSpec((D,), lambda i: (0,)),
            pl.BlockSpec((D, D), lambda i: (0, 0)),
            pl.BlockSpec((D,), lambda i: (0,)),
        ],
        out_specs=pl.BlockSpec((2, _FB, 64), lambda i: (0, i, 0)),
        out_shape=jax.ShapeDtypeStruct((2, N, 64), _F32),
    )(s1o, s1o, s1o, s1o, s23o, s23o, s23o, s23o,
      scal, scal, x, W_dgn, b_dgn, t1w, t1b)


_YRPS = NHE // 16
_XRPS = N // 16


_HSLOT = 32


def _hg_pipe(sid, v_hbm, e_hbm, src_hbm, dst_sp, goff, gather_by_e,
             iv, ie, igat, buf, sg, sm, ss):
    def cond(k):
        return sid + 16 * k < _HNC

    def meta_issue(k, st):
        base = (sid + 16 * k) * _HK
        pltpu.async_copy(v_hbm.at[pl.ds(base, _HK)], iv.at[st], sm[st])
        pltpu.async_copy(e_hbm.at[pl.ds(base, _HK)], ie.at[st], sm[st])

    def meta_wait(k, st):
        base = (sid + 16 * k) * _HK
        pltpu.make_async_copy(v_hbm.at[pl.ds(base, _HK)], iv.at[st], sm[st]).wait()
        pltpu.make_async_copy(e_hbm.at[pl.ds(base, _HK)], ie.at[st], sm[st]).wait()

    def buf_sl(st):
        return buf.at[pl.ds(st * _HK, _HK)]

    def build_igat(st):
        gref = ie if gather_by_e else iv
        for j in range(0, _HK, 16):
            igat[st, pl.ds(j, 16)] = gref[st, pl.ds(j, 16)] + goff

    def gather_issue(st):
        pltpu.async_copy(src_hbm.at[igat.at[st]], buf_sl(st), sg[st])

    def gather_wait(st):
        pltpu.make_async_copy(src_hbm.at[igat.at[st]], buf_sl(st), sg[st]).wait()

    def sidx(st):
        return iv.at[st] if gather_by_e else ie.at[st]

    def scatter_issue(st):
        pltpu.async_copy(buf_sl(st), dst_sp.at[sidx(st)], ss[st], add=True)

    def scatter_wait(st):
        pltpu.make_async_copy(buf_sl(st), dst_sp.at[sidx(st)], ss[st]).wait()

    meta_issue(0, 0)
    meta_wait(0, 0)
    build_igat(0)
    gather_issue(0)
    meta_issue(1, 1)

    @pl.loop(0, _HSLOT // 4)
    def _(kk):
        for ci in range(4):
            k = 4 * kk + ci

            @pl.when(jnp.logical_and(k >= 2, cond(k - 2)))
            def _():
                scatter_wait((ci + 2) % 4)

            @pl.when(cond(k + 2))
            def _():
                meta_issue(k + 2, (ci + 2) % 4)

            @pl.when(jnp.logical_and(k + 1 < _HSLOT, cond(k + 1)))
            def _():
                meta_wait(k + 1, (ci + 1) % 4)
                build_igat((ci + 1) % 4)
                gather_issue((ci + 1) % 4)

            @pl.when(cond(k))
            def _():
                gather_wait(ci)
                scatter_issue(ci)

    for k in (_HSLOT - 2, _HSLOT - 1):
        @pl.when(cond(k))
        def _():
            scatter_wait(k % 4)


def _hg_body(relu,
             xp_hbm, xn_hbm, pv_hbm, pe_hbm, nv_hbm, ne_hbm,
             iep_hbm, ivp_hbm, ien_hbm, ivn_hbm,
             xop_hbm, xon_hbm, yp_hbm, yn_hbm,
             yp_sp, yn_sp, xop_sp, xon_sp,
             iv, ie, igat, buf, invb, zb,
             sg0, sg1, sg2, sg3, sm0, sm1, sm2, sm3, ss0, ss1, ss2, ss3):
    c = lax.axis_index("c")
    sid = lax.axis_index("s")
    sg = (sg0, sg1, sg2, sg3)
    sm = (sm0, sm1, sm2, sm3)
    ss = (ss0, ss1, ss2, ss3)
    _zero_vmem(zb, _RC, 64)
    ry = sid * _YRPS
    rx = sid * _XRPS

    pltpu.sync_copy(zb.at[pl.ds(0, _YRPS)], yp_sp.at[pl.ds(ry, _YRPS)])
    pltpu.sync_copy(zb.at[pl.ds(0, _YRPS)], yn_sp.at[pl.ds(ry, _YRPS)])

    @pl.loop(0, 5)
    def _(k):
        rr = rx + k * _RC
        pltpu.sync_copy(zb, xop_sp.at[pl.ds(rr, _RC)])
        pltpu.sync_copy(zb, xon_sp.at[pl.ds(rr, _RC)])

    plsc.subcore_barrier()

    coff = c * N

    _hg_pipe(sid, pv_hbm, pe_hbm, xp_hbm, yp_sp, coff, False,
             iv, ie, igat, buf, sg, sm, ss)
    _hg_pipe(sid, nv_hbm, ne_hbm, xn_hbm, yn_sp, coff, False,
             iv, ie, igat, buf, sg, sm, ss)

    plsc.subcore_barrier()

    for (ie_hbm, y_sp, y_hbm) in ((iep_hbm, yp_sp, yp_hbm),
                                  (ien_hbm, yn_sp, yn_hbm)):
        pltpu.sync_copy(y_sp.at[pl.ds(ry, _YRPS)], zb.at[pl.ds(0, _YRPS)])
        pltpu.sync_copy(ie_hbm.at[pl.ds(ry, _YRPS)], invb)

        @pl.loop(0, _YRPS)
        def _(r):
            fi = invb[r, pl.ds(0, 16)]
            for f in range(4):
                zb[r, pl.ds(f * 16, 16)] = zb[r, pl.ds(f * 16, 16)] * fi

        pltpu.sync_copy(zb.at[pl.ds(0, _YRPS)], y_hbm.at[pl.ds(c * NHE + ry, _YRPS)])

    plsc.subcore_barrier()

    yoff = c * NHE
    _hg_pipe(sid, pv_hbm, pe_hbm, yp_hbm, xop_sp, yoff, True,
             iv, ie, igat, buf, sg, sm, ss)
    _hg_pipe(sid, nv_hbm, ne_hbm, yn_hbm, xon_sp, yoff, True,
             iv, ie, igat, buf, sg, sm, ss)

    plsc.subcore_barrier()

    for (iv_hbm, xo_sp, xo_hbm) in ((ivp_hbm, xop_sp, xop_hbm),
                                    (ivn_hbm, xon_sp, xon_hbm)):
        @pl.loop(0, 5)
        def _(k):
            rr = rx + k * _RC
            pltpu.sync_copy(xo_sp.at[pl.ds(rr, _RC)], zb)
            pltpu.sync_copy(iv_hbm.at[pl.ds(rr, _RC)], invb)

            @pl.loop(0, _RC)
            def _(r):
                fi = invb[r, pl.ds(0, 16)]
                for f in range(4):
                    val = zb[r, pl.ds(f * 16, 16)] * fi
                    if relu:
                        val = jnp.maximum(val, 0.0)
                    zb[r, pl.ds(f * 16, 16)] = val

            pltpu.sync_copy(zb, xo_hbm.at[pl.ds(coff + rr, _RC)])


def _hg_sc(relu, xp, xn, pv, pe, nv, ne, iep, ivp, ien, ivn):
    def body(*refs):
        return _hg_body(relu, *refs)

    fn = pl.kernel(
        body,
        out_type=(
            jax.ShapeDtypeStruct((2 * N, 64), _F32),
            jax.ShapeDtypeStruct((2 * N, 64), _F32),
            jax.ShapeDtypeStruct((2 * NHE, 64), _F32),
            jax.ShapeDtypeStruct((2 * NHE, 64), _F32),
        ),
        mesh=_SC_MESH,
        scratch_types=[
            pltpu.VMEM_SHARED((NHE, 64), _F32),
            pltpu.VMEM_SHARED((NHE, 64), _F32),
            pltpu.VMEM_SHARED((N, 64), _F32),
            pltpu.VMEM_SHARED((N, 64), _F32),
            pltpu.VMEM((4, _HK), _I32),
            pltpu.VMEM((4, _HK), _I32),
            pltpu.VMEM((4, _HK), _I32),
            pltpu.VMEM((4 * _HK, 64), _F32),
            pltpu.VMEM((_RC, 16), _F32),
            pltpu.VMEM((_RC, 64), _F32),
        ] + [pltpu.SemaphoreType.DMA] * 12,
        compiler_params=_SC_PARAMS,
    )
    return fn(xp, xn, pv, pe, nv, ne, iep, ivp, ien, ivn)


def _theta2_body(xpa, xpb, xna, xnb, w_ref, b_ref, op_ref, on_ref):
    w = w_ref[...]
    b = b_ref[...]
    Xp = jnp.concatenate([xpa[...], xpb[...]], axis=-1)
    Xn = jnp.concatenate([xna[...], xnb[...]], axis=-1)
    Yp = jnp.dot(Xp, w, preferred_element_type=_F32) + b
    Yn = jnp.dot(Xn, w, preferred_element_type=_F32) + b
    op_ref[...] = jnp.stack([Yp[:, :64], Yp[:, 64:]])
    on_ref[...] = jnp.stack([Yn[:, :64], Yn[:, 64:]])


def _theta2_tc(xop, xon, t2w, t2b):
    grid = N // _FB
    off = N // _FB
    return pl.pallas_call(
        _theta2_body,
        grid=(grid,),
        in_specs=[
            pl.BlockSpec((_FB, 64), lambda i: (i, 0)),
            pl.BlockSpec((_FB, 64), lambda i: (i + off, 0)),
            pl.BlockSpec((_FB, 64), lambda i: (i, 0)),
            pl.BlockSpec((_FB, 64), lambda i: (i + off, 0)),
            pl.BlockSpec((D, D), lambda i: (0, 0)),
            pl.BlockSpec((D,), lambda i: (0,)),
        ],
        out_specs=[
            pl.BlockSpec((2, _FB, 64), lambda i: (0, i, 0)),
            pl.BlockSpec((2, _FB, 64), lambda i: (0, i, 0)),
        ],
        out_shape=[
            jax.ShapeDtypeStruct((2, N, 64), _F32),
            jax.ShapeDtypeStruct((2, N, 64), _F32),
        ],
    )(xop, xop, xon, xon, t2w, t2b)


def kernel(m_emb, edge_index, eig, hg_pos_v, hg_pos_e, hg_neg_v, hg_neg_e,
           W1, b1, W2, b2, W3, b3, W_dgn, b_dgn,
           theta1_W, theta1_b, theta2_W, theta2_b):
    src = edge_index[0].astype(_I32)
    dst = edge_index[1].astype(_I32)
    eig2 = jnp.asarray(eig[:, 2], _F32)
    pv = hg_pos_v.astype(_I32)
    pe = hg_pos_e.astype(_I32)
    nv = hg_neg_v.astype(_I32)
    ne = hg_neg_e.astype(_I32)

    ea, ee, scal, cvp, cep, cvn, cen = _prep_sc(src, dst, eig2, pv, pe, nv, ne)
    x, xq = _mlp_tc(m_emb, W1, b1, W2, b2, W3, b3)
    xq2 = xq.reshape(4 * N, 32)

    s1o, s23o = _dgn_sc(xq2, src, dst, ea, ee)
    Xs = _finish_tc(s1o, s23o, scal, x, W_dgn, b_dgn,
                    theta1_W, theta1_b)
    Xflat = Xs.reshape(2 * N, 64)
    ivp, iep, ivn, ien = _inv_tc(cvp, cep, cvn, cen)

    xop, xon, _, _ = _hg_sc(True, Xflat, Xflat, pv, pe, nv, ne,
                            iep, ivp, ien, ivn)
    Xps, Xns = _theta2_tc(xop, xon, theta2_W, theta2_b)
    xop2, xon2, yp2, yn2 = _hg_sc(False, Xps.reshape(2 * N, 64),
                                  Xns.reshape(2 * N, 64), pv, pe, nv, ne,
                                  iep, ivp, ien, ivn)

    X1 = jnp.concatenate([xop2[:N], xop2[N:]], axis=-1)
    X2 = jnp.concatenate([xon2[:N], xon2[N:]], axis=-1)
    Yp = jnp.concatenate([yp2[:NHE], yp2[NHE:]], axis=-1)
    Yn = jnp.concatenate([yn2[:NHE], yn2[NHE:]], axis=-1)
    return (X1, X2, Yp, Yn)
